# Initial kernel scaffold; baseline (speedup 1.0000x reference)
import jax
import jax.numpy as jnp
from jax.experimental import pallas as pl


def kernel(x, table):
    raise NotImplementedError("write your pallas kernel here")


# speedup vs baseline: 6.1855x; 6.1855x over previous
import functools

import jax
import jax.numpy as jnp
from jax import lax
from jax.experimental import pallas as pl
from jax.experimental.pallas import tpu as pltpu
from jax.experimental.pallas import tpu_sc as plsc

D = 128
CHUNK = 128


@functools.cache
def _make_sc_gather(N):
    info = plsc.get_sparse_core_info()
    NC, NS = info.num_cores, info.num_subcores
    NW = NC * NS
    n_per_w = N // NW
    n_chunks = n_per_w // CHUNK
    assert n_per_w * NW == N and n_chunks * CHUNK == n_per_w

    mesh = plsc.VectorSubcoreMesh(core_axis_name="c", subcore_axis_name="s")

    @functools.partial(
        pl.kernel,
        mesh=mesh,
        out_type=jax.ShapeDtypeStruct((N, D), jnp.float32),
        scratch_types=[
            pltpu.VMEM((n_chunks, CHUNK), jnp.int32),
            pltpu.VMEM((CHUNK, D), jnp.float32),
            pltpu.SemaphoreType.DMA,
        ],
    )
    def gather_kernel(x_hbm, table_hbm, out_hbm, idx_v, rows_v, sem):
        wid = lax.axis_index("s") * NC + lax.axis_index("c")
        base = wid * n_per_w
        pltpu.sync_copy(x_hbm.at[wid], idx_v)

        def body(j, carry):
            pltpu.async_copy(table_hbm.at[idx_v.at[j]], rows_v, sem).wait()
            pltpu.sync_copy(rows_v, out_hbm.at[pl.ds(base + j * CHUNK, CHUNK)])
            return carry

        lax.fori_loop(0, n_chunks, body, 0)

    return gather_kernel


def kernel(x, table):
    B, L = x.shape
    N = B * L
    info = plsc.get_sparse_core_info()
    NW = info.num_cores * info.num_subcores
    xf = x.astype(jnp.int32).reshape(NW, N // (NW * CHUNK), CHUNK)
    out = _make_sc_gather(N)(xf, table)
    return out.reshape(B, L, D)[:, None]


# speedup vs baseline: 6.6539x; 1.0757x over previous
import functools

import jax
import jax.numpy as jnp
from jax import lax
from jax.experimental import pallas as pl
from jax.experimental.pallas import tpu as pltpu
from jax.experimental.pallas import tpu_sc as plsc

D = 128
CHUNK = 128


@functools.cache
def _make_sc_gather(N):
    info = plsc.get_sparse_core_info()
    NC, NS = info.num_cores, info.num_subcores
    NW = NC * NS
    n_per_w = N // NW
    n_chunks = n_per_w // CHUNK
    assert n_per_w * NW == N and n_chunks * CHUNK == n_per_w

    mesh = plsc.VectorSubcoreMesh(core_axis_name="c", subcore_axis_name="s")

    @functools.partial(
        pl.kernel,
        mesh=mesh,
        out_type=jax.ShapeDtypeStruct((N, D), jnp.float32),
        scratch_types=[
            pltpu.VMEM((n_chunks, CHUNK), jnp.int32),
            pltpu.VMEM((CHUNK, D), jnp.float32),
            pltpu.VMEM((CHUNK, D), jnp.float32),
            pltpu.SemaphoreType.DMA,
            pltpu.SemaphoreType.DMA,
            pltpu.SemaphoreType.DMA,
            pltpu.SemaphoreType.DMA,
        ],
    )
    def gather_kernel(x_hbm, table_hbm, out_hbm, idx_v, b0, b1,
                      gsem0, gsem1, osem0, osem1):
        wid = lax.axis_index("s") * NC + lax.axis_index("c")
        base = wid * n_per_w
        pltpu.sync_copy(x_hbm.at[wid], idx_v)

        def gather(j, buf, sem):
            pltpu.async_copy(table_hbm.at[idx_v.at[j]], buf, sem)

        def gather_wait(j, buf, sem):
            pltpu.make_async_copy(table_hbm.at[idx_v.at[j]], buf, sem).wait()

        gather(0, b0, gsem0)

        def body(i, carry):
            j = 2 * i
            gather(j + 1, b1, gsem1)
            gather_wait(j, b0, gsem0)
            w0 = pltpu.async_copy(
                b0, out_hbm.at[pl.ds(base + j * CHUNK, CHUNK)], osem0)
            gather_wait(j + 1, b1, gsem1)
            w0.wait()

            @pl.when(j + 2 < n_chunks)
            def _():
                gather(j + 2, b0, gsem0)

            w1 = pltpu.async_copy(
                b1, out_hbm.at[pl.ds(base + (j + 1) * CHUNK, CHUNK)], osem1)
            w1.wait()
            return carry

        lax.fori_loop(0, n_chunks // 2, body, 0)

    return gather_kernel


def kernel(x, table):
    B, L = x.shape
    N = B * L
    info = plsc.get_sparse_core_info()
    NW = info.num_cores * info.num_subcores
    xf = x.astype(jnp.int32).reshape(NW, N // (NW * CHUNK), CHUNK)
    out = _make_sc_gather(N)(xf, table)
    return out.reshape(B, L, D)[:, None]


# speedup vs baseline: 15.2985x; 2.2992x over previous
import functools

import jax
import jax.numpy as jnp
from jax import lax
from jax.experimental import pallas as pl
from jax.experimental.pallas import tpu as pltpu
from jax.experimental.pallas import tpu_sc as plsc

VOCAB = 1000
D = 128
CHUNK = 128


@functools.cache
def _make_sc_gather(N):
    info = plsc.get_sparse_core_info()
    NC, NS = info.num_cores, info.num_subcores
    NW = NC * NS
    n_per_w = N // NW
    n_chunks = n_per_w // CHUNK
    assert n_per_w * NW == N and n_chunks * CHUNK == n_per_w

    mesh = plsc.VectorSubcoreMesh(core_axis_name="c", subcore_axis_name="s")

    @functools.partial(
        pl.kernel,
        mesh=mesh,
        out_type=jax.ShapeDtypeStruct((N, D), jnp.float32),
        scratch_types=[
            pltpu.MemorySpace.VMEM_SHARED((VOCAB, D), jnp.float32),
            pltpu.VMEM((n_chunks, CHUNK), jnp.int32),
            pltpu.VMEM((CHUNK, D), jnp.float32),
            pltpu.VMEM((CHUNK, D), jnp.float32),
            pltpu.SemaphoreType.DMA,
            pltpu.SemaphoreType.DMA,
            pltpu.SemaphoreType.DMA,
            pltpu.SemaphoreType.DMA,
        ],
    )
    def gather_kernel(x_hbm, table_hbm, out_hbm, table_sh, idx_v, b0, b1,
                      gsem0, gsem1, osem0, osem1):
        sid = lax.axis_index("s")
        wid = sid * NC + lax.axis_index("c")
        base = wid * n_per_w

        @pl.when(sid == 0)
        def _():
            pltpu.sync_copy(table_hbm, table_sh)

        pltpu.sync_copy(x_hbm.at[wid], idx_v)
        plsc.subcore_barrier()

        def gather(j, buf, sem):
            pltpu.async_copy(table_sh.at[idx_v.at[j]], buf, sem)

        def gather_wait(j, buf, sem):
            pltpu.make_async_copy(table_sh.at[idx_v.at[j]], buf, sem).wait()

        gather(0, b0, gsem0)

        def body(i, carry):
            j = 2 * i
            gather(j + 1, b1, gsem1)
            gather_wait(j, b0, gsem0)
            w0 = pltpu.async_copy(
                b0, out_hbm.at[pl.ds(base + j * CHUNK, CHUNK)], osem0)
            gather_wait(j + 1, b1, gsem1)
            w0.wait()

            @pl.when(j + 2 < n_chunks)
            def _():
                gather(j + 2, b0, gsem0)

            w1 = pltpu.async_copy(
                b1, out_hbm.at[pl.ds(base + (j + 1) * CHUNK, CHUNK)], osem1)
            w1.wait()
            return carry

        lax.fori_loop(0, n_chunks // 2, body, 0)

    return gather_kernel


def kernel(x, table):
    B, L = x.shape
    N = B * L
    info = plsc.get_sparse_core_info()
    NW = info.num_cores * info.num_subcores
    xf = x.astype(jnp.int32).reshape(NW, N // (NW * CHUNK), CHUNK)
    out = _make_sc_gather(N)(xf, table)
    return out.reshape(B, L, D)[:, None]


# speedup vs baseline: 15.5210x; 1.0145x over previous
import functools

import jax
import jax.numpy as jnp
from jax import lax
from jax.experimental import pallas as pl
from jax.experimental.pallas import tpu as pltpu
from jax.experimental.pallas import tpu_sc as plsc

VOCAB = 1024
D = 128
CHUNK = 128


@functools.cache
def _make_sc_gather(N):
    info = plsc.get_sparse_core_info()
    NC, NS = info.num_cores, info.num_subcores
    NW = NC * NS
    n_per_w = N // NW
    n_chunks = n_per_w // CHUNK
    assert n_per_w * NW == N and n_chunks * CHUNK == n_per_w

    mesh = plsc.VectorSubcoreMesh(core_axis_name="c", subcore_axis_name="s")

    NBUF = 4
    assert n_chunks % NBUF == 0 and VOCAB % (NS // 2) == 0

    @functools.partial(
        pl.kernel,
        mesh=mesh,
        out_type=jax.ShapeDtypeStruct((N, D), jnp.float32),
        scratch_types=[
            pltpu.MemorySpace.VMEM_SHARED((VOCAB, D), jnp.float32),
            pltpu.VMEM((n_chunks, CHUNK), jnp.int32),
        ]
        + [pltpu.VMEM((CHUNK, D), jnp.float32)] * NBUF
        + [pltpu.SemaphoreType.DMA] * (2 * NBUF),
    )
    def gather_kernel(x_hbm, table_hbm, out_hbm, table_sh, idx_v, *rest):
        bufs = rest[:NBUF]
        gsem = rest[NBUF:2 * NBUF]
        osem = rest[2 * NBUF:]
        sid = lax.axis_index("s")
        wid = sid * NC + lax.axis_index("c")
        base = wid * n_per_w

        rows = VOCAB // (NS // 2)

        @pl.when(sid < NS // 2)
        def _():
            pltpu.sync_copy(table_hbm.at[pl.ds(sid * rows, rows)],
                            table_sh.at[pl.ds(sid * rows, rows)])

        pltpu.sync_copy(x_hbm.at[wid], idx_v)
        plsc.subcore_barrier()

        def gather(j, b):
            pltpu.async_copy(table_sh.at[idx_v.at[j]], bufs[b], gsem[b])

        def gather_wait(j, b):
            pltpu.make_async_copy(
                table_sh.at[idx_v.at[j]], bufs[b], gsem[b]).wait()

        def write(j, b):
            pltpu.async_copy(
                bufs[b], out_hbm.at[pl.ds(base + j * CHUNK, CHUNK)], osem[b])

        def write_wait(j, b):
            pltpu.make_async_copy(
                bufs[b], out_hbm.at[pl.ds(base + j * CHUNK, CHUNK)],
                osem[b]).wait()

        for b in range(NBUF):
            gather(b, b)

        def body(i, carry):
            for b in range(NBUF):
                j = NBUF * i + b
                gather_wait(j, b)
                write(j, b)

                @pl.when(j + NBUF < n_chunks)
                def _():
                    write_wait(j, b)
                    gather(j + NBUF, b)

            return carry

        lax.fori_loop(0, n_chunks // NBUF, body, 0)
        for b in range(NBUF):
            write_wait(n_chunks - NBUF + b, b)

    return gather_kernel


def kernel(x, table):
    B, L = x.shape
    N = B * L
    info = plsc.get_sparse_core_info()
    NW = info.num_cores * info.num_subcores
    xf = x.astype(jnp.int32).reshape(NW, N // (NW * CHUNK), CHUNK)
    tpad = jnp.zeros((VOCAB, D), table.dtype).at[:table.shape[0]].set(table)
    out = _make_sc_gather(N)(xf, tpad)
    return out.reshape(B, L, D)[:, None]


# speedup vs baseline: 15.5478x; 1.0017x over previous
import functools

import jax
import jax.numpy as jnp
from jax import lax
from jax.experimental import pallas as pl
from jax.experimental.pallas import tpu as pltpu
from jax.experimental.pallas import tpu_sc as plsc

VOCAB = 1024
D = 128
CHUNK = 128


@functools.cache
def _make_sc_gather(N):
    info = plsc.get_sparse_core_info()
    NC, NS = info.num_cores, info.num_subcores
    NW = NC * NS
    n_per_w = N // NW
    n_chunks = n_per_w // CHUNK
    assert n_per_w * NW == N and n_chunks * CHUNK == n_per_w

    mesh = plsc.VectorSubcoreMesh(core_axis_name="c", subcore_axis_name="s")

    NBUF = 2
    n_pairs = n_chunks // 2
    assert n_pairs % NBUF == 0 and VOCAB % (NS // 2) == 0

    @functools.partial(
        pl.kernel,
        mesh=mesh,
        out_type=jax.ShapeDtypeStruct((N, D), jnp.float32),
        scratch_types=[
            pltpu.MemorySpace.VMEM_SHARED((VOCAB, D), jnp.float32),
            pltpu.VMEM((n_chunks, CHUNK), jnp.int32),
        ]
        + [pltpu.VMEM((2 * CHUNK, D), jnp.float32)] * NBUF
        + [pltpu.SemaphoreType.DMA] * (3 * NBUF),
    )
    def gather_kernel(x_hbm, table_hbm, out_hbm, table_sh, idx_v, *rest):
        bufs = rest[:NBUF]
        gsem = rest[NBUF:3 * NBUF]
        osem = rest[3 * NBUF:]
        sid = lax.axis_index("s")
        wid = sid * NC + lax.axis_index("c")
        base = wid * n_per_w

        rows = VOCAB // (NS // 2)

        @pl.when(sid < NS // 2)
        def _():
            pltpu.sync_copy(table_hbm.at[pl.ds(sid * rows, rows)],
                            table_sh.at[pl.ds(sid * rows, rows)])

        pltpu.sync_copy(x_hbm.at[wid], idx_v)
        plsc.subcore_barrier()

        def gather(p, b):
            for h in range(2):
                pltpu.async_copy(
                    table_sh.at[idx_v.at[2 * p + h]],
                    bufs[b].at[pl.ds(h * CHUNK, CHUNK)], gsem[2 * b + h])

        def gather_wait(p, b):
            for h in range(2):
                pltpu.make_async_copy(
                    table_sh.at[idx_v.at[2 * p + h]],
                    bufs[b].at[pl.ds(h * CHUNK, CHUNK)],
                    gsem[2 * b + h]).wait()

        def write(p, b):
            pltpu.async_copy(
                bufs[b], out_hbm.at[pl.ds(base + p * 2 * CHUNK, 2 * CHUNK)],
                osem[b])

        def write_wait(p, b):
            pltpu.make_async_copy(
                bufs[b], out_hbm.at[pl.ds(base + p * 2 * CHUNK, 2 * CHUNK)],
                osem[b]).wait()

        for b in range(NBUF):
            gather(b, b)

        def body(i, carry):
            for b in range(NBUF):
                p = NBUF * i + b
                gather_wait(p, b)
                write(p, b)

                @pl.when(p + NBUF < n_pairs)
                def _():
                    write_wait(p, b)
                    gather(p + NBUF, b)

            return carry

        lax.fori_loop(0, n_pairs // NBUF, body, 0)
        for b in range(NBUF):
            write_wait(n_pairs - NBUF + b, b)

    return gather_kernel


def kernel(x, table):
    B, L = x.shape
    N = B * L
    info = plsc.get_sparse_core_info()
    NW = info.num_cores * info.num_subcores
    xf = x.astype(jnp.int32).reshape(NW, N // (NW * CHUNK), CHUNK)
    tpad = jnp.zeros((VOCAB, D), table.dtype).at[:table.shape[0]].set(table)
    out = _make_sc_gather(N)(xf, tpad)
    return out.reshape(B, L, D)[:, None]


# speedup vs baseline: 15.6760x; 1.0082x over previous
import functools

import jax
import jax.numpy as jnp
from jax import lax
from jax.experimental import pallas as pl
from jax.experimental.pallas import tpu as pltpu
from jax.experimental.pallas import tpu_sc as plsc

VOCAB = 1024
D = 128
CHUNK = 128


@functools.cache
def _make_sc_gather(N):
    info = plsc.get_sparse_core_info()
    NC, NS = info.num_cores, info.num_subcores
    NW = NC * NS
    n_per_w = N // NW
    n_chunks = n_per_w // CHUNK
    assert n_per_w * NW == N and n_chunks * CHUNK == n_per_w

    mesh = plsc.VectorSubcoreMesh(core_axis_name="c", subcore_axis_name="s")

    NBUF = 2
    n_pairs = n_chunks // 2
    assert n_pairs % NBUF == 0 and VOCAB % NS == 0

    @functools.partial(
        pl.kernel,
        mesh=mesh,
        out_type=jax.ShapeDtypeStruct((N, D), jnp.float32),
        scratch_types=[
            pltpu.MemorySpace.VMEM_SHARED((VOCAB, D), jnp.float32),
            pltpu.VMEM((n_chunks, CHUNK), jnp.int32),
        ]
        + [pltpu.VMEM((2 * CHUNK, D), jnp.float32)] * NBUF
        + [pltpu.SemaphoreType.DMA] * (3 * NBUF + 2),
    )
    def gather_kernel(x_hbm, table_hbm, out_hbm, table_sh, idx_v, *rest):
        bufs = rest[:NBUF]
        gsem = rest[NBUF:3 * NBUF]
        osem = rest[3 * NBUF:3 * NBUF + NBUF]
        tsem, isem = rest[3 * NBUF + NBUF:]
        sid = lax.axis_index("s")
        wid = sid * NC + lax.axis_index("c")
        base = wid * n_per_w

        rows = VOCAB // NS
        tab_copy = pltpu.make_async_copy(
            table_hbm.at[pl.ds(sid * rows, rows)],
            table_sh.at[pl.ds(sid * rows, rows)], tsem)
        idx_copy = pltpu.make_async_copy(x_hbm.at[wid], idx_v, isem)
        tab_copy.start()
        idx_copy.start()
        tab_copy.wait()
        idx_copy.wait()
        plsc.subcore_barrier()

        def gather(p, b):
            for h in range(2):
                pltpu.async_copy(
                    table_sh.at[idx_v.at[2 * p + h]],
                    bufs[b].at[pl.ds(h * CHUNK, CHUNK)], gsem[2 * b + h])

        def gather_wait(p, b):
            for h in range(2):
                pltpu.make_async_copy(
                    table_sh.at[idx_v.at[2 * p + h]],
                    bufs[b].at[pl.ds(h * CHUNK, CHUNK)],
                    gsem[2 * b + h]).wait()

        def write(p, b):
            pltpu.async_copy(
                bufs[b], out_hbm.at[pl.ds(base + p * 2 * CHUNK, 2 * CHUNK)],
                osem[b])

        def write_wait(p, b):
            pltpu.make_async_copy(
                bufs[b], out_hbm.at[pl.ds(base + p * 2 * CHUNK, 2 * CHUNK)],
                osem[b]).wait()

        for b in range(NBUF):
            gather(b, b)

        def body(i, carry):
            for b in range(NBUF):
                p = NBUF * i + b
                gather_wait(p, b)
                write(p, b)

                @pl.when(p + NBUF < n_pairs)
                def _():
                    write_wait(p, b)
                    gather(p + NBUF, b)

            return carry

        lax.fori_loop(0, n_pairs // NBUF, body, 0)
        for b in range(NBUF):
            write_wait(n_pairs - NBUF + b, b)

    return gather_kernel


def kernel(x, table):
    B, L = x.shape
    N = B * L
    info = plsc.get_sparse_core_info()
    NW = info.num_cores * info.num_subcores
    xf = x.astype(jnp.int32).reshape(NW, N // (NW * CHUNK), CHUNK)
    tpad = jnp.zeros((VOCAB, D), table.dtype).at[:table.shape[0]].set(table)
    out = _make_sc_gather(N)(xf, tpad)
    return out.reshape(B, L, D)[:, None]
